# baseline (speedup 1.0000x reference)
import functools

import jax
import jax.numpy as jnp
from jax import lax
from jax.experimental import pallas as pl
from jax.experimental.pallas import tpu as pltpu
from jax.experimental.pallas import tpu_sc as plsc

NPAD = 10240
BN = 512
NSUB = 16
CHP = 64
CH_H = 40
SETS = 5


def _sc_hist(col, ones, zeros):
    e = col.shape[0]
    pt = e // (2 * NSUB)
    nch = pt // CH_H
    rt = NPAD // NSUB
    mesh = plsc.VectorSubcoreMesh(core_axis_name="c", subcore_axis_name="s")

    @functools.partial(
        pl.kernel,
        out_type=jax.ShapeDtypeStruct((2, NPAD), jnp.float32),
        mesh=mesh,
        scratch_types=(
            [pltpu.VMEM((CH_H,), jnp.int32) for _ in range(SETS)]
            + [pltpu.VMEM((CH_H,), jnp.float32),
               pltpu.VMEM_SHARED((NPAD,), jnp.float32)]
            + [pltpu.SemaphoreType.DMA] * (2 * SETS)
        ),
    )
    def hist_k(col_hbm, ones_hbm, zeros_hbm, out_hbm, *scr):
        cidx = scr[0:SETS]
        ones_v = scr[SETS]
        acc = scr[SETS + 1]
        isem = scr[SETS + 2:SETS + 2 + SETS]
        ssem = scr[SETS + 2 + SETS:]
        c = lax.axis_index("c")
        s = lax.axis_index("s")
        w = s * 2 + c
        base = w * pt

        def fetch(k_, r):
            off = pl.multiple_of(base + k_ * CH_H, 8)
            return pltpu.async_copy(col_hbm.at[pl.ds(off, CH_H)],
                                    cidx[r], isem[r])

        def fetch_wait(k_, r):
            off = pl.multiple_of(base + k_ * CH_H, 8)
            pltpu.make_async_copy(col_hbm.at[pl.ds(off, CH_H)],
                                  cidx[r], isem[r]).wait()

        def scat_wait(r):
            pltpu.make_async_copy(ones_v, acc.at[cidx[r]], ssem[r]).wait()

        fetch(0, 0)
        fetch(1, 1)
        pltpu.sync_copy(zeros_hbm.at[pl.ds(s * rt, rt)],
                        acc.at[pl.ds(s * rt, rt)])
        pltpu.sync_copy(ones_hbm, ones_v)
        plsc.subcore_barrier()

        def chunk(k_, r):
            fetch_wait(k_, r)
            pltpu.async_copy(ones_v, acc.at[cidx[r]], ssem[r], add=True)
            r2 = (r + 2) % SETS

            @pl.when(k_ >= 3)
            def _():
                scat_wait(r2)

            @pl.when(k_ + 2 < nch)
            def _():
                fetch(k_ + 2, r2)

        def body(m, carry):
            for r in range(SETS):
                chunk(SETS * m + r, r)
            return carry

        lax.fori_loop(0, nch // SETS, body, 0)
        for kk in (nch - 3, nch - 2, nch - 1):
            scat_wait(kk % SETS)
        plsc.subcore_barrier()
        pltpu.sync_copy(acc.at[pl.ds(s * rt, rt)],
                        out_hbm.at[c, pl.ds(s * rt, rt)])

    return hist_k(col, ones, zeros)


def _sc_agg(table, packed, zeros):
    dh = table.shape[1]
    _, ncht, _, chp = packed.shape
    npt = ncht // NSUB
    rt = NPAD // NSUB
    mesh = plsc.VectorSubcoreMesh(core_axis_name="c", subcore_axis_name="s")

    @functools.partial(
        pl.kernel,
        out_type=jax.ShapeDtypeStruct((2, NPAD, dh), jnp.float32),
        mesh=mesh,
        scratch_types=(
            [pltpu.VMEM((2, chp), jnp.int32) for _ in range(SETS)]
            + [pltpu.VMEM((chp, dh), jnp.float32) for _ in range(SETS)]
            + [pltpu.VMEM_SHARED((NPAD + 8, dh), jnp.float32)]
            + [pltpu.SemaphoreType.DMA] * (3 * SETS)
        ),
    )
    def agg_k(table_hbm, packed_hbm, zeros_hbm, out_hbm, *scr):
        eidx = scr[0:SETS]
        msg = scr[SETS:2 * SETS]
        acc = scr[2 * SETS]
        isem = scr[2 * SETS + 1:2 * SETS + 1 + SETS]
        gsem = scr[2 * SETS + 1 + SETS:2 * SETS + 1 + 2 * SETS]
        ssem = scr[2 * SETS + 1 + 2 * SETS:]
        c = lax.axis_index("c")
        s = lax.axis_index("s")
        base = s * npt

        def fetch(k_, r):
            pltpu.async_copy(packed_hbm.at[c, base + k_], eidx[r], isem[r])

        def fetch_wait(k_, r):
            pltpu.make_async_copy(packed_hbm.at[c, base + k_],
                                  eidx[r], isem[r]).wait()

        def gath(r):
            pltpu.async_copy(table_hbm.at[eidx[r].at[0]], msg[r], gsem[r])

        def gath_wait(r):
            pltpu.make_async_copy(table_hbm.at[eidx[r].at[0]], msg[r],
                                  gsem[r]).wait()

        def scat_wait(r):
            pltpu.make_async_copy(msg[r], acc.at[eidx[r].at[1]],
                                  ssem[r]).wait()

        fetch(0, 0)
        fetch(1, 1)
        fetch(2, 2)
        pltpu.sync_copy(zeros_hbm.at[pl.ds(s * rt, rt), :],
                        acc.at[pl.ds(s * rt, rt), :])
        fetch_wait(0, 0)
        gath(0)
        plsc.subcore_barrier()

        def chunk(k_, r):
            r1 = (r + 1) % SETS
            r3 = (r + 3) % SETS
            gath_wait(r)
            pltpu.async_copy(msg[r], acc.at[eidx[r].at[1]], ssem[r],
                             add=True)

            @pl.when(k_ >= 2)
            def _():
                scat_wait(r3)

            @pl.when(k_ + 3 < npt)
            def _():
                fetch(k_ + 3, r3)

            @pl.when(k_ + 1 < npt)
            def _():
                fetch_wait(k_ + 1, r1)
                gath(r1)

        def body(m, carry):
            for r in range(SETS):
                chunk(SETS * m + r, r)
            return carry

        lax.fori_loop(0, npt // SETS, body, 0)
        for kk in (npt - 2, npt - 1):
            scat_wait(kk % SETS)
        plsc.subcore_barrier()
        pltpu.sync_copy(acc.at[pl.ds(s * rt, rt), :],
                        out_hbm.at[c, pl.ds(s * rt, rt), :])

    return agg_k(table, packed, zeros)


def _dinv_block(h_blk, pid, n_real):
    cnt = h_blk[0] + h_blk[1]
    rowid = pid * BN + lax.broadcasted_iota(jnp.int32, (BN, 1), 0)
    return jnp.where(rowid < n_real, lax.rsqrt(cnt + 1.0), 0.0)


def _tc_first(nodes_p, w, hist, n_real):
    do = w.shape[0]
    dh = do // 2
    d = nodes_p.shape[1]

    def k(x_ref, w_ref, h_ref, o_ref):
        pid = pl.program_id(0)
        dinv = _dinv_block(h_ref[...], pid, n_real)
        z = lax.dot_general(x_ref[...], w_ref[...], (((1,), (1,)), ((), ())),
                            preferred_element_type=jnp.float32)
        zd = dinv * z
        o_ref[0] = zd[:, :dh]
        o_ref[1] = zd[:, dh:]

    return pl.pallas_call(
        k,
        grid=(NPAD // BN,),
        in_specs=[
            pl.BlockSpec((BN, d), lambda i: (i, 0)),
            pl.BlockSpec((do, d), lambda i: (0, 0)),
            pl.BlockSpec((2, BN, 1), lambda i: (0, i, 0)),
        ],
        out_specs=pl.BlockSpec((2, BN, dh), lambda i: (0, i, 0)),
        out_shape=jax.ShapeDtypeStruct((2, NPAD, dh), jnp.float32),
    )(nodes_p, w, hist)


def _tc_mid(agg, zs_v, b2, w, hist, n_real):
    dh = zs_v.shape[2]
    do = w.shape[0]
    split = do > 128
    dho = do // 2

    def k(a_ref, z_ref, b_ref, w_ref, h_ref, o_ref):
        pid = pl.program_id(0)
        dinv = _dinv_block(h_ref[...], pid, n_real)
        a = a_ref[...]
        zz = z_ref[...]
        bb = b_ref[...]
        x0 = dinv * (a[0] + zz[0]) + bb[0]
        x1 = dinv * (a[1] + zz[1]) + bb[1]
        z = lax.dot_general(x0, w_ref[:, :dh], (((1,), (1,)), ((), ())),
                            preferred_element_type=jnp.float32)
        z += lax.dot_general(x1, w_ref[:, dh:], (((1,), (1,)), ((), ())),
                             preferred_element_type=jnp.float32)
        zd = dinv * z
        if split:
            o_ref[0] = zd[:, :dho]
            o_ref[1] = zd[:, dho:]
        else:
            o_ref[...] = zd

    if split:
        out_spec = pl.BlockSpec((2, BN, dho), lambda i: (0, i, 0))
        out_shape = jax.ShapeDtypeStruct((2, NPAD, dho), jnp.float32)
    else:
        out_spec = pl.BlockSpec((BN, do), lambda i: (i, 0))
        out_shape = jax.ShapeDtypeStruct((NPAD, do), jnp.float32)
    return pl.pallas_call(
        k,
        grid=(NPAD // BN,),
        in_specs=[
            pl.BlockSpec((2, BN, dh), lambda i: (0, i, 0)),
            pl.BlockSpec((2, BN, dh), lambda i: (0, i, 0)),
            pl.BlockSpec((2, dh), lambda i: (0, 0)),
            pl.BlockSpec((do, 2 * dh), lambda i: (0, 0)),
            pl.BlockSpec((2, BN, 1), lambda i: (0, i, 0)),
        ],
        out_specs=out_spec,
        out_shape=out_shape,
    )(agg, zs_v, b2, w, hist)


def _tc_final(agg, zs, b3, hist, n_real):
    dh = zs.shape[1]

    def k(a_ref, z_ref, b_ref, h_ref, o_ref):
        pid = pl.program_id(0)
        dinv = _dinv_block(h_ref[...], pid, n_real)
        a = a_ref[...]
        o_ref[...] = dinv * (a[0] + a[1] + z_ref[...]) + b_ref[...]

    return pl.pallas_call(
        k,
        grid=(NPAD // BN,),
        in_specs=[
            pl.BlockSpec((2, BN, dh), lambda i: (0, i, 0)),
            pl.BlockSpec((BN, dh), lambda i: (i, 0)),
            pl.BlockSpec((1, dh), lambda i: (0, 0)),
            pl.BlockSpec((2, BN, 1), lambda i: (0, i, 0)),
        ],
        out_specs=pl.BlockSpec((BN, dh), lambda i: (i, 0)),
        out_shape=jax.ShapeDtypeStruct((NPAD, dh), jnp.float32),
    )(agg, zs, b3, hist)


def kernel(nodes, edges, W1, b1, W2, b2, W3, b3):
    n, d = nodes.shape
    row = edges[0]
    col = edges[1]

    nodes_p = jnp.pad(nodes, ((0, NPAD - n), (0, 0)))
    ones_h = jnp.ones((CH_H,), jnp.float32)
    zeros_h = jnp.zeros((NPAD,), jnp.float32)
    zeros_t = jnp.zeros((NPAD, 128), jnp.float32)

    e = row.shape[0]

    def pack(r_, c_, parts):
        per = r_.shape[1]
        padded = ((per + SETS * CHP - 1) // (SETS * CHP)) * (SETS * CHP)
        rp = jnp.pad(r_, ((0, 0), (0, padded - per)))
        cp = jnp.pad(c_, ((0, 0), (0, padded - per)),
                     constant_values=NPAD)
        rp = rp.reshape(parts, padded // CHP, CHP)
        cp = cp.reshape(parts, padded // CHP, CHP)
        return jnp.stack([rp, cp], axis=2)

    pf = pack(row.reshape(NSUB, -1), col.reshape(NSUB, -1), NSUB)
    pf = pf.reshape(-1, 2, CHP)
    off1 = jnp.stack([jnp.full((CHP,), NPAD, jnp.int32),
                      jnp.zeros((CHP,), jnp.int32)])
    p_fs = jnp.stack([pf, pf + off1])
    pe = pack(row.reshape(2 * NSUB, -1), col.reshape(2 * NSUB, -1), 2 * NSUB)
    p_es = pe.reshape(2, -1, 2, CHP)

    hist = _sc_hist(col, ones_h, zeros_h).reshape(2, NPAD, 1)

    w2p = jnp.pad(W2, ((0, 256 - W2.shape[0]), (0, 0)))
    b1v = b1.reshape(2, 128)
    b2p = jnp.pad(b2, (0, 256 - b2.shape[0])).reshape(2, 128)
    w3p = jnp.pad(W3, ((0, 0), (0, 256 - W3.shape[1])))

    zs1 = _tc_first(nodes_p, W1, hist, n)
    agg1 = _sc_agg(zs1.reshape(2 * NPAD, 128), p_fs, zeros_t)

    zs2 = _tc_mid(agg1, zs1, b1v, w2p, hist, n)
    agg2 = _sc_agg(zs2.reshape(2 * NPAD, 128), p_fs, zeros_t)

    zs3 = _tc_mid(agg2, zs2, b2p, w3p, hist, n)
    agg3 = _sc_agg(zs3, p_es, zeros_t)

    y = _tc_final(agg3, zs3, b3.reshape(1, 128), hist, n)
    return y[:n]


# speedup vs baseline: 1.9984x; 1.9984x over previous
import functools

import jax
import jax.numpy as jnp
from jax import lax
from jax.experimental import pallas as pl
from jax.experimental.pallas import tpu as pltpu
from jax.experimental.pallas import tpu_sc as plsc

NPAD = 10240
BN = 512
NSUB = 16
CHP = 40
CH_H = 40
SETS = 5


def _sc_hist(col, ones, zeros):
    e = col.shape[0]
    pt = e // (2 * NSUB)
    nch = pt // CH_H
    rt = NPAD // NSUB
    mesh = plsc.VectorSubcoreMesh(core_axis_name="c", subcore_axis_name="s")

    @functools.partial(
        pl.kernel,
        out_type=jax.ShapeDtypeStruct((2, NPAD), jnp.float32),
        mesh=mesh,
        scratch_types=(
            [pltpu.VMEM((CH_H,), jnp.int32) for _ in range(SETS)]
            + [pltpu.VMEM((CH_H,), jnp.float32),
               pltpu.VMEM_SHARED((NPAD,), jnp.float32)]
            + [pltpu.SemaphoreType.DMA] * (2 * SETS)
        ),
    )
    def hist_k(col_hbm, ones_hbm, zeros_hbm, out_hbm, *scr):
        cidx = scr[0:SETS]
        ones_v = scr[SETS]
        acc = scr[SETS + 1]
        isem = scr[SETS + 2:SETS + 2 + SETS]
        ssem = scr[SETS + 2 + SETS:]
        c = lax.axis_index("c")
        s = lax.axis_index("s")
        w = s * 2 + c
        base = w * pt

        def fetch(k_, r):
            off = pl.multiple_of(base + k_ * CH_H, 8)
            return pltpu.async_copy(col_hbm.at[pl.ds(off, CH_H)],
                                    cidx[r], isem[r])

        def fetch_wait(k_, r):
            off = pl.multiple_of(base + k_ * CH_H, 8)
            pltpu.make_async_copy(col_hbm.at[pl.ds(off, CH_H)],
                                  cidx[r], isem[r]).wait()

        def scat_wait(r):
            pltpu.make_async_copy(ones_v, acc.at[cidx[r]], ssem[r]).wait()

        fetch(0, 0)
        fetch(1, 1)
        pltpu.sync_copy(zeros_hbm.at[pl.ds(s * rt, rt)],
                        acc.at[pl.ds(s * rt, rt)])
        pltpu.sync_copy(ones_hbm, ones_v)
        plsc.subcore_barrier()

        def chunk(k_, r):
            fetch_wait(k_, r)
            pltpu.async_copy(ones_v, acc.at[cidx[r]], ssem[r], add=True)
            r2 = (r + 2) % SETS

            @pl.when(k_ >= 3)
            def _():
                scat_wait(r2)

            @pl.when(k_ + 2 < nch)
            def _():
                fetch(k_ + 2, r2)

        def body(m, carry):
            for r in range(SETS):
                chunk(SETS * m + r, r)
            return carry

        lax.fori_loop(0, nch // SETS, body, 0)
        for kk in (nch - 3, nch - 2, nch - 1):
            scat_wait(kk % SETS)
        plsc.subcore_barrier()
        pltpu.sync_copy(acc.at[pl.ds(s * rt, rt)],
                        out_hbm.at[c, pl.ds(s * rt, rt)])

    return hist_k(col, ones, zeros)


def _sc_agg(table, packed, zeros):
    dh = table.shape[1]
    _, ncht, _, chp = packed.shape
    npt = ncht // NSUB
    rt = NPAD // NSUB
    mesh = plsc.VectorSubcoreMesh(core_axis_name="c", subcore_axis_name="s")

    @functools.partial(
        pl.kernel,
        out_type=jax.ShapeDtypeStruct((2, NPAD, dh), jnp.float32),
        mesh=mesh,
        scratch_types=(
            [pltpu.VMEM((2, chp), jnp.int32) for _ in range(SETS)]
            + [pltpu.VMEM((chp, dh), jnp.float32) for _ in range(SETS)]
            + [pltpu.VMEM_SHARED((NPAD + 8, dh), jnp.float32)]
            + [pltpu.SemaphoreType.DMA] * (3 * SETS)
        ),
    )
    def agg_k(table_hbm, packed_hbm, zeros_hbm, out_hbm, *scr):
        eidx = scr[0:SETS]
        msg = scr[SETS:2 * SETS]
        acc = scr[2 * SETS]
        isem = scr[2 * SETS + 1:2 * SETS + 1 + SETS]
        gsem = scr[2 * SETS + 1 + SETS:2 * SETS + 1 + 2 * SETS]
        ssem = scr[2 * SETS + 1 + 2 * SETS:]
        c = lax.axis_index("c")
        s = lax.axis_index("s")
        base = s * npt

        def fetch(k_, r):
            pltpu.async_copy(packed_hbm.at[c, base + k_], eidx[r], isem[r])

        def fetch_wait(k_, r):
            pltpu.make_async_copy(packed_hbm.at[c, base + k_],
                                  eidx[r], isem[r]).wait()

        def gath(r):
            pltpu.async_copy(table_hbm.at[eidx[r].at[0]], msg[r], gsem[r])

        def gath_wait(r):
            pltpu.make_async_copy(table_hbm.at[eidx[r].at[0]], msg[r],
                                  gsem[r]).wait()

        def scat_wait(r):
            pltpu.make_async_copy(msg[r], acc.at[eidx[r].at[1]],
                                  ssem[r]).wait()

        fetch(0, 0)
        fetch(1, 1)
        fetch(2, 2)
        pltpu.sync_copy(zeros_hbm.at[pl.ds(s * rt, rt), :],
                        acc.at[pl.ds(s * rt, rt), :])
        fetch_wait(0, 0)
        gath(0)
        fetch_wait(1, 1)
        gath(1)
        plsc.subcore_barrier()

        def chunk(k_, r):
            r2 = (r + 2) % SETS
            r3 = (r + 3) % SETS
            gath_wait(r)
            pltpu.async_copy(msg[r], acc.at[eidx[r].at[1]], ssem[r],
                             add=True)

            @pl.when(k_ >= 2)
            def _():
                scat_wait(r3)

            @pl.when(k_ + 3 < npt)
            def _():
                fetch(k_ + 3, r3)

            @pl.when(k_ + 2 < npt)
            def _():
                fetch_wait(k_ + 2, r2)
                gath(r2)

        def body(m, carry):
            for r in range(SETS):
                chunk(SETS * m + r, r)
            return carry

        lax.fori_loop(0, npt // SETS, body, 0)
        for kk in (npt - 2, npt - 1):
            scat_wait(kk % SETS)
        plsc.subcore_barrier()
        pltpu.sync_copy(acc.at[pl.ds(s * rt, rt), :],
                        out_hbm.at[c, pl.ds(s * rt, rt), :])

    return agg_k(table, packed, zeros)


def _dinv_block(h_blk, pid, n_real):
    cnt = h_blk[0] + h_blk[1]
    rowid = pid * BN + lax.broadcasted_iota(jnp.int32, (BN, 1), 0)
    return jnp.where(rowid < n_real, lax.rsqrt(cnt + 1.0), 0.0)


def _tc_first(nodes_p, w, hist, n_real):
    do = w.shape[0]
    dh = do // 2
    d = nodes_p.shape[1]

    def k(x_ref, w_ref, h_ref, o_ref):
        pid = pl.program_id(0)
        dinv = _dinv_block(h_ref[...], pid, n_real)
        z = lax.dot_general(x_ref[...], w_ref[...], (((1,), (1,)), ((), ())),
                            preferred_element_type=jnp.float32)
        zd = dinv * z
        o_ref[0] = zd[:, :dh]
        o_ref[1] = zd[:, dh:]

    return pl.pallas_call(
        k,
        grid=(NPAD // BN,),
        in_specs=[
            pl.BlockSpec((BN, d), lambda i: (i, 0)),
            pl.BlockSpec((do, d), lambda i: (0, 0)),
            pl.BlockSpec((2, BN, 1), lambda i: (0, i, 0)),
        ],
        out_specs=pl.BlockSpec((2, BN, dh), lambda i: (0, i, 0)),
        out_shape=jax.ShapeDtypeStruct((2, NPAD, dh), jnp.float32),
    )(nodes_p, w, hist)


def _tc_mid(agg, zs_v, b2, w, hist, n_real):
    dh = zs_v.shape[2]
    do = w.shape[0]
    split = do > 128
    dho = do // 2

    def k(a_ref, z_ref, b_ref, w_ref, h_ref, o_ref):
        pid = pl.program_id(0)
        dinv = _dinv_block(h_ref[...], pid, n_real)
        a = a_ref[...]
        zz = z_ref[...]
        bb = b_ref[...]
        x0 = dinv * (a[0] + zz[0]) + bb[0]
        x1 = dinv * (a[1] + zz[1]) + bb[1]
        z = lax.dot_general(x0, w_ref[:, :dh], (((1,), (1,)), ((), ())),
                            preferred_element_type=jnp.float32)
        z += lax.dot_general(x1, w_ref[:, dh:], (((1,), (1,)), ((), ())),
                             preferred_element_type=jnp.float32)
        zd = dinv * z
        if split:
            o_ref[0] = zd[:, :dho]
            o_ref[1] = zd[:, dho:]
        else:
            o_ref[...] = zd

    if split:
        out_spec = pl.BlockSpec((2, BN, dho), lambda i: (0, i, 0))
        out_shape = jax.ShapeDtypeStruct((2, NPAD, dho), jnp.float32)
    else:
        out_spec = pl.BlockSpec((BN, do), lambda i: (i, 0))
        out_shape = jax.ShapeDtypeStruct((NPAD, do), jnp.float32)
    return pl.pallas_call(
        k,
        grid=(NPAD // BN,),
        in_specs=[
            pl.BlockSpec((2, BN, dh), lambda i: (0, i, 0)),
            pl.BlockSpec((2, BN, dh), lambda i: (0, i, 0)),
            pl.BlockSpec((2, dh), lambda i: (0, 0)),
            pl.BlockSpec((do, 2 * dh), lambda i: (0, 0)),
            pl.BlockSpec((2, BN, 1), lambda i: (0, i, 0)),
        ],
        out_specs=out_spec,
        out_shape=out_shape,
    )(agg, zs_v, b2, w, hist)


def _tc_final(agg, zs, b3, hist, n_real):
    dh = zs.shape[1]

    def k(a_ref, z_ref, b_ref, h_ref, o_ref):
        pid = pl.program_id(0)
        dinv = _dinv_block(h_ref[...], pid, n_real)
        a = a_ref[...]
        o_ref[...] = dinv * (a[0] + a[1] + z_ref[...]) + b_ref[...]

    return pl.pallas_call(
        k,
        grid=(NPAD // BN,),
        in_specs=[
            pl.BlockSpec((2, BN, dh), lambda i: (0, i, 0)),
            pl.BlockSpec((BN, dh), lambda i: (i, 0)),
            pl.BlockSpec((1, dh), lambda i: (0, 0)),
            pl.BlockSpec((2, BN, 1), lambda i: (0, i, 0)),
        ],
        out_specs=pl.BlockSpec((BN, dh), lambda i: (i, 0)),
        out_shape=jax.ShapeDtypeStruct((NPAD, dh), jnp.float32),
    )(agg, zs, b3, hist)


def kernel(nodes, edges, W1, b1, W2, b2, W3, b3):
    n, d = nodes.shape
    row = edges[0]
    col = edges[1]

    nodes_p = jnp.pad(nodes, ((0, NPAD - n), (0, 0)))
    ones_h = jnp.ones((CH_H,), jnp.float32)
    zeros_h = jnp.zeros((NPAD,), jnp.float32)
    zeros_t = jnp.zeros((NPAD, 128), jnp.float32)

    e = row.shape[0]

    def pack(r_, c_, parts):
        per = r_.shape[1]
        padded = ((per + SETS * CHP - 1) // (SETS * CHP)) * (SETS * CHP)
        rp = jnp.pad(r_, ((0, 0), (0, padded - per)))
        cp = jnp.pad(c_, ((0, 0), (0, padded - per)),
                     constant_values=NPAD)
        rp = rp.reshape(parts, padded // CHP, CHP)
        cp = cp.reshape(parts, padded // CHP, CHP)
        return jnp.stack([rp, cp], axis=2)

    pf = pack(row.reshape(NSUB, -1), col.reshape(NSUB, -1), NSUB)
    pf = pf.reshape(-1, 2, CHP)
    off1 = jnp.stack([jnp.full((CHP,), NPAD, jnp.int32),
                      jnp.zeros((CHP,), jnp.int32)])
    p_fs = jnp.stack([pf, pf + off1])
    pe = pack(row.reshape(2 * NSUB, -1), col.reshape(2 * NSUB, -1), 2 * NSUB)
    p_es = pe.reshape(2, -1, 2, CHP)

    hist = _sc_hist(col, ones_h, zeros_h).reshape(2, NPAD, 1)

    w2p = jnp.pad(W2, ((0, 256 - W2.shape[0]), (0, 0)))
    b1v = b1.reshape(2, 128)
    b2p = jnp.pad(b2, (0, 256 - b2.shape[0])).reshape(2, 128)
    w3p = jnp.pad(W3, ((0, 0), (0, 256 - W3.shape[1])))

    zs1 = _tc_first(nodes_p, W1, hist, n)
    agg1 = _sc_agg(zs1.reshape(2 * NPAD, 128), p_fs, zeros_t)

    zs2 = _tc_mid(agg1, zs1, b1v, w2p, hist, n)
    agg2 = _sc_agg(zs2.reshape(2 * NPAD, 128), p_fs, zeros_t)

    zs3 = _tc_mid(agg2, zs2, b2p, w3p, hist, n)
    agg3 = _sc_agg(zs3, p_es, zeros_t)

    y = _tc_final(agg3, zs3, b3.reshape(1, 128), hist, n)
    return y[:n]
